# baseline (speedup 1.0000x reference)
import jax
import jax.numpy as jnp
from jax.experimental import pallas as pl

NC = 1000
ROWS = 4096 * 26
BLK_R = 512


def _onehot_body(x_ref, out_ref):
    iota = jax.lax.broadcasted_iota(jnp.int32, (BLK_R, NC), 1)
    out_ref[:, :] = (iota == x_ref[:, :]).astype(jnp.float32)


def kernel(x):
    xf = x.reshape(ROWS, 1).astype(jnp.int32)
    out = pl.pallas_call(
        _onehot_body,
        grid=(ROWS // BLK_R,),
        in_specs=[pl.BlockSpec((BLK_R, 1), lambda i: (i, 0))],
        out_specs=pl.BlockSpec((BLK_R, NC), lambda i: (i, 0)),
        out_shape=jax.ShapeDtypeStruct((ROWS, NC), jnp.float32),
    )(xf)
    return out.reshape(4096, 26, NC)


# speedup vs baseline: 1.5575x; 1.5575x over previous
import jax
import jax.numpy as jnp
from jax.experimental import pallas as pl

NC = 1000
B0 = 4096
B1 = 26
BLK = 128


def _onehot_body(x_ref, out_ref):
    iota = jax.lax.broadcasted_iota(jnp.int32, (BLK, B1, NC), 2)
    xv = x_ref[:, :]
    out_ref[:, :, :] = (iota == xv[:, :, None]).astype(jnp.float32)


def kernel(x):
    xi = x.astype(jnp.int32)
    out = pl.pallas_call(
        _onehot_body,
        grid=(B0 // BLK,),
        in_specs=[pl.BlockSpec((BLK, B1), lambda i: (i, 0))],
        out_specs=pl.BlockSpec((BLK, B1, NC), lambda i: (i, 0, 0)),
        out_shape=jax.ShapeDtypeStruct((B0, B1, NC), jnp.float32),
    )(xi)
    return out


# speedup vs baseline: 1.5577x; 1.0001x over previous
import jax
import jax.numpy as jnp
from jax.experimental import pallas as pl
from jax.experimental.pallas import tpu as pltpu

NC = 1000
B0 = 4096
B1 = 26
CB = 32
NBUF = 4
NSTEP = B0 // CB


def _onehot_body(x_ref, out_ref, scratch, sems):
    i = pl.program_id(0)
    j = jax.lax.rem(i, NBUF)
    b0 = i * CB

    def mkcopy(jj):
        return pltpu.make_async_copy(
            scratch.at[jj], out_ref.at[pl.ds(b0, CB)], sems.at[jj]
        )

    @pl.when(i >= NBUF)
    def _wait_prev():
        mkcopy(j).wait()

    iota = jax.lax.broadcasted_iota(jnp.int32, (CB, B1, NC), 2)
    xv = x_ref[:, :]
    scratch[j] = (iota == xv[:, :, None]).astype(jnp.float32)
    mkcopy(j).start()

    @pl.when(i == NSTEP - 1)
    def _drain():
        for jj in range(NBUF):
            mkcopy(jj).wait()


def kernel(x):
    xi = x.astype(jnp.int32)
    out = pl.pallas_call(
        _onehot_body,
        grid=(NSTEP,),
        in_specs=[pl.BlockSpec((CB, B1), lambda i: (i, 0))],
        out_specs=pl.BlockSpec(memory_space=pl.ANY),
        out_shape=jax.ShapeDtypeStruct((B0, B1, NC), jnp.float32),
        scratch_shapes=[
            pltpu.VMEM((NBUF, CB, B1, NC), jnp.float32),
            pltpu.SemaphoreType.DMA((NBUF,)),
        ],
    )(xi)
    return out


# speedup vs baseline: 7.2466x; 4.6521x over previous
import jax
import jax.numpy as jnp
from jax.experimental import pallas as pl

NC = 1000
B0 = 4096
B1 = 26


def _onehot_body(x_ref, out_ref):
    xv = x_ref[:, :, :]
    iota = jax.lax.broadcasted_iota(jnp.int32, (1, NC, B0), 1)
    out_ref[:, :, :] = (iota == xv).astype(jnp.float32)


def kernel(x):
    xt = x.astype(jnp.int32).T.reshape(B1, 1, B0)
    out_t = pl.pallas_call(
        _onehot_body,
        grid=(B1,),
        in_specs=[pl.BlockSpec((1, 1, B0), lambda j: (j, 0, 0))],
        out_specs=pl.BlockSpec((1, NC, B0), lambda j: (j, 0, 0)),
        out_shape=jax.ShapeDtypeStruct((B1, NC, B0), jnp.float32),
    )(xt)
    return jnp.transpose(out_t, (2, 0, 1))
